# baseline (speedup 1.0000x reference)
import functools

import jax
import jax.numpy as jnp
from jax import lax
from jax.experimental import pallas as pl
from jax.experimental.pallas import tpu as pltpu
from jax.experimental.pallas import tpu_sc as plsc

NUM_CORES = 2
NUM_SUBCORES = 16
BLK = 128
CHUNK = 8
HALF = 128
PAD_ROWS = 8
LANES = 16

_MESH = plsc.VectorSubcoreMesh(
    core_axis_name="c", subcore_axis_name="s",
    num_cores=NUM_CORES, num_subcores=NUM_SUBCORES)


def _stripes(n_dst):
    stripe = ((n_dst + NUM_SUBCORES - 1) // NUM_SUBCORES + 7) // 8 * 8
    return stripe, n_dst - (NUM_SUBCORES - 1) * stripe


def _sc_gather_scatter(x_stk, esrc2d, edst2d, n_dst):
    n_src = x_stk.shape[0] // NUM_CORES
    n_blocks = esrc2d.shape[0]
    blocks_per_sub = n_blocks // NUM_SUBCORES
    n_chunks = blocks_per_sub // CHUNK
    n_acc = n_dst + PAD_ROWS
    stripe, last_stripe = _stripes(n_dst)

    zf = jnp.zeros((n_dst, HALF), jnp.float32)

    @functools.partial(
        pl.kernel,
        out_type=(
            jax.ShapeDtypeStruct((n_dst, HALF), jnp.float32),
            jax.ShapeDtypeStruct((n_dst, HALF), jnp.float32),
        ),
        mesh=_MESH,
        scratch_types=[
            pltpu.VMEM((CHUNK, BLK), jnp.int32),
            pltpu.VMEM((CHUNK, BLK), jnp.int32),
            pltpu.VMEM((BLK, HALF), jnp.float32),
            pltpu.VMEM((BLK, HALF), jnp.float32),
            pltpu.VMEM_SHARED((n_acc, HALF), jnp.float32),
            pltpu.SemaphoreType.DMA,
            pltpu.SemaphoreType.DMA,
        ],
    )
    def k(x_hbm, es_hbm, ed_hbm, zf_hbm,
          agg0_hbm, agg1_hbm,
          idx_s, idx_d, rows0, rows1, acc, sem0, sem1):
        c = lax.axis_index("c")
        s = lax.axis_index("s")

        def on_stripe(fn):
            @pl.when(s < NUM_SUBCORES - 1)
            def _():
                fn(s * stripe, stripe)

            @pl.when(s == NUM_SUBCORES - 1)
            def _():
                fn((NUM_SUBCORES - 1) * stripe, last_stripe)

        blk0 = s * blocks_per_sub
        on_stripe(lambda r0, nr: pltpu.sync_copy(
            zf_hbm.at[pl.ds(r0, nr)], acc.at[pl.ds(r0, nr)]))
        plsc.subcore_barrier()

        bufs = ((rows0, sem0), (rows1, sem1))

        def gather(j, rows, sem):
            return pltpu.make_async_copy(x_hbm.at[idx_s.at[j]], rows, sem)

        @pl.loop(0, n_chunks)
        def _(m):
            pltpu.sync_copy(
                es_hbm.at[c].at[pl.ds(blk0 + m * CHUNK, CHUNK)], idx_s)
            pltpu.sync_copy(ed_hbm.at[pl.ds(blk0 + m * CHUNK, CHUNK)], idx_d)

            gather(0, rows0, sem0).start()
            for j in range(CHUNK):
                if j + 1 < CHUNK:
                    nrows, nsem = bufs[(j + 1) % 2]
                    gather(j + 1, nrows, nsem).start()
                rows, sem = bufs[j % 2]
                gather(j, rows, sem).wait()
                pltpu.sync_copy(rows, acc.at[idx_d.at[j]], add=True)

        plsc.subcore_barrier()

        @pl.when(c == 0)
        def _():
            on_stripe(lambda r0, nr: pltpu.sync_copy(
                acc.at[pl.ds(r0, nr)], agg0_hbm.at[pl.ds(r0, nr)]))

        @pl.when(c == 1)
        def _():
            on_stripe(lambda r0, nr: pltpu.sync_copy(
                acc.at[pl.ds(r0, nr)], agg1_hbm.at[pl.ds(r0, nr)]))

    es_stk = jnp.stack([esrc2d, esrc2d + jnp.int32(n_src)])
    return k(x_stk, es_stk, edst2d, zf)


def _sc_degree(edst2d, n_dst):
    n_blocks = edst2d.shape[0]
    blocks_per_w = n_blocks // (NUM_CORES * NUM_SUBCORES)
    n_chunks = blocks_per_w // CHUNK
    n_acc = n_dst + PAD_ROWS
    stripe, last_stripe = _stripes(n_dst)

    zd = jnp.zeros((n_dst, HALF), jnp.float32)
    ones = jnp.ones((BLK, HALF), jnp.float32)

    @functools.partial(
        pl.kernel,
        out_type=(
            jax.ShapeDtypeStruct((n_dst, HALF), jnp.float32),
            jax.ShapeDtypeStruct((n_dst, HALF), jnp.float32),
        ),
        mesh=_MESH,
        scratch_types=[
            pltpu.VMEM((CHUNK, BLK), jnp.int32),
            pltpu.VMEM((BLK, HALF), jnp.float32),
            pltpu.VMEM_SHARED((n_acc, HALF), jnp.float32),
        ],
    )
    def k(ed_hbm, zd_hbm, ones_hbm, dega_hbm, degb_hbm,
          idx_d, ones_v, dacc):
        c = lax.axis_index("c")
        s = lax.axis_index("s")

        def on_stripe(fn):
            @pl.when(s < NUM_SUBCORES - 1)
            def _():
                fn(s * stripe, stripe)

            @pl.when(s == NUM_SUBCORES - 1)
            def _():
                fn((NUM_SUBCORES - 1) * stripe, last_stripe)

        blk0 = (c * NUM_SUBCORES + s) * blocks_per_w
        on_stripe(lambda r0, nr: pltpu.sync_copy(
            zd_hbm.at[pl.ds(r0, nr)], dacc.at[pl.ds(r0, nr)]))
        pltpu.sync_copy(ones_hbm, ones_v)
        plsc.subcore_barrier()

        @pl.loop(0, n_chunks)
        def _(m):
            pltpu.sync_copy(ed_hbm.at[pl.ds(blk0 + m * CHUNK, CHUNK)], idx_d)
            for j in range(CHUNK):
                pltpu.sync_copy(ones_v, dacc.at[idx_d.at[j]], add=True)

        plsc.subcore_barrier()

        @pl.when(c == 0)
        def _():
            on_stripe(lambda r0, nr: pltpu.sync_copy(
                dacc.at[pl.ds(r0, nr)], dega_hbm.at[pl.ds(r0, nr)]))

        @pl.when(c == 1)
        def _():
            on_stripe(lambda r0, nr: pltpu.sync_copy(
                dacc.at[pl.ds(r0, nr)], degb_hbm.at[pl.ds(r0, nr)]))

    return k(edst2d, zd, ones)


def _tc_mean_matmul(agg0, agg1, dega, degb, x_dst, w_nbr, w_self):
    n, d = x_dst.shape
    blk = 1000
    wn0 = w_nbr[:HALF]
    wn1 = w_nbr[HALF:]

    def body(a0_ref, a1_ref, da_ref, db_ref, x_ref,
             wn0_ref, wn1_ref, ws_ref, o_ref):
        deg = da_ref[:, 0:1] + db_ref[:, 0:1]
        r = 1.0 / jnp.maximum(deg, 1.0)
        o_ref[...] = (
            jnp.dot(a0_ref[...] * r, wn0_ref[...],
                    preferred_element_type=jnp.float32)
            + jnp.dot(a1_ref[...] * r, wn1_ref[...],
                      preferred_element_type=jnp.float32)
            + jnp.dot(x_ref[...], ws_ref[...],
                      preferred_element_type=jnp.float32)
        )

    return pl.pallas_call(
        body,
        grid=(n // blk,),
        in_specs=[
            pl.BlockSpec((blk, HALF), lambda i: (i, 0)),
            pl.BlockSpec((blk, HALF), lambda i: (i, 0)),
            pl.BlockSpec((blk, HALF), lambda i: (i, 0)),
            pl.BlockSpec((blk, HALF), lambda i: (i, 0)),
            pl.BlockSpec((blk, d), lambda i: (i, 0)),
            pl.BlockSpec((HALF, d), lambda i: (0, 0)),
            pl.BlockSpec((HALF, d), lambda i: (0, 0)),
            pl.BlockSpec((d, d), lambda i: (0, 0)),
        ],
        out_specs=pl.BlockSpec((blk, d), lambda i: (i, 0)),
        out_shape=jax.ShapeDtypeStruct((n, d), jnp.float32),
    )(agg0, agg1, dega, degb, x_dst, wn0, wn1, w_self)


def _pad_edges(e_src, e_dst, n_dst):
    e = e_src.shape[0]
    chunk = NUM_SUBCORES * BLK * CHUNK
    e_pad = ((e + chunk - 1) // chunk) * chunk
    npad = e_pad - e
    e_src = jnp.concatenate(
        [e_src.astype(jnp.int32), jnp.zeros((npad,), jnp.int32)])
    e_dst = jnp.concatenate(
        [e_dst.astype(jnp.int32), jnp.full((npad,), n_dst, jnp.int32)])
    return e_src.reshape(-1, BLK), e_dst.reshape(-1, BLK)


def _conv(x_src, x_dst, e_src, e_dst, w_nbr, w_self):
    n_dst = x_dst.shape[0]
    es2d, ed2d = _pad_edges(e_src, e_dst, n_dst)
    x_stk = jnp.concatenate([x_src[:, :HALF], x_src[:, HALF:]], axis=0)
    agg0, agg1 = _sc_gather_scatter(x_stk, es2d, ed2d, n_dst)
    dega, degb = _sc_degree(ed2d, n_dst)
    return _tc_mean_matmul(agg0, agg1, dega, degb, x_dst, w_nbr, w_self)


def kernel(x_user, x_item, u2i_src, u2i_dst, i2u_src, i2u_dst,
           n_user, n_item, W_nbr_u2i, W_self_u2i, W_nbr_i2u, W_self_i2u):
    out_item = _conv(x_user, x_item, u2i_src, u2i_dst, W_nbr_u2i, W_self_u2i)
    out_user = _conv(x_item, x_user, i2u_src, i2u_dst, W_nbr_i2u, W_self_i2u)
    return (out_user, out_item)


# speedup vs baseline: 1.0606x; 1.0606x over previous
import functools

import jax
import jax.numpy as jnp
from jax import lax
from jax.experimental import pallas as pl
from jax.experimental.pallas import tpu as pltpu
from jax.experimental.pallas import tpu_sc as plsc

NUM_CORES = 2
NUM_SUBCORES = 16
BLK = 128
CHUNK = 8
HALF = 128
PAD_ROWS = 8
LANES = 16

_MESH = plsc.VectorSubcoreMesh(
    core_axis_name="c", subcore_axis_name="s",
    num_cores=NUM_CORES, num_subcores=NUM_SUBCORES)


def _stripes(n_dst):
    stripe = ((n_dst + NUM_SUBCORES - 1) // NUM_SUBCORES + 7) // 8 * 8
    return stripe, n_dst - (NUM_SUBCORES - 1) * stripe


def _sc_gather_scatter(x0, x1, esrc2d, edst2d, n_dst):
    n_blocks = esrc2d.shape[0]
    blocks_per_sub = n_blocks // NUM_SUBCORES
    n_chunks = blocks_per_sub // CHUNK
    n_acc = n_dst + PAD_ROWS
    stripe, last_stripe = _stripes(n_dst)

    zf = jnp.zeros((n_dst, HALF), jnp.float32)

    @functools.partial(
        pl.kernel,
        out_type=(
            jax.ShapeDtypeStruct((n_dst, HALF), jnp.float32),
            jax.ShapeDtypeStruct((n_dst, HALF), jnp.float32),
        ),
        mesh=_MESH,
        scratch_types=[
            pltpu.VMEM((CHUNK, BLK), jnp.int32),
            pltpu.VMEM((CHUNK, BLK), jnp.int32),
            pltpu.VMEM((BLK, HALF), jnp.float32),
            pltpu.VMEM((BLK, HALF), jnp.float32),
            pltpu.VMEM_SHARED((n_acc, HALF), jnp.float32),
            pltpu.SemaphoreType.DMA,
            pltpu.SemaphoreType.DMA,
            pltpu.SemaphoreType.DMA,
            pltpu.SemaphoreType.DMA,
        ],
    )
    def k(x0_hbm, x1_hbm, es_hbm, ed_hbm, zf_hbm,
          agg0_hbm, agg1_hbm,
          idx_s, idx_d, rows0, rows1, acc, sem0, sem1, ssem0, ssem1):
        c = lax.axis_index("c")
        s = lax.axis_index("s")

        def on_stripe(fn):
            @pl.when(s < NUM_SUBCORES - 1)
            def _():
                fn(s * stripe, stripe)

            @pl.when(s == NUM_SUBCORES - 1)
            def _():
                fn((NUM_SUBCORES - 1) * stripe, last_stripe)

        blk0 = s * blocks_per_sub
        on_stripe(lambda r0, nr: pltpu.sync_copy(
            zf_hbm.at[pl.ds(r0, nr)], acc.at[pl.ds(r0, nr)]))
        plsc.subcore_barrier()

        bufs = ((rows0, sem0, ssem0), (rows1, sem1, ssem1))

        def main_loop(x_hbm):
            def gather(j, rows, sem):
                return pltpu.make_async_copy(x_hbm.at[idx_s.at[j]],
                                             rows, sem)

            class scatter:
                def __init__(self, j):
                    self.rows, _, self.ssem = bufs[j % 2]
                    self.j = j

                def start(self):
                    pltpu.async_copy(self.rows, acc.at[idx_d.at[self.j]],
                                     self.ssem, add=True)

                def wait(self):
                    pltpu.make_async_copy(self.rows,
                                          acc.at[idx_d.at[self.j]],
                                          self.ssem).wait()

            @pl.loop(0, n_chunks)
            def _(m):
                pltpu.sync_copy(es_hbm.at[pl.ds(blk0 + m * CHUNK, CHUNK)],
                                idx_s)
                pltpu.sync_copy(ed_hbm.at[pl.ds(blk0 + m * CHUNK, CHUNK)],
                                idx_d)

                @pl.when(m > 0)
                def _():
                    scatter(CHUNK - 2).wait()

                gather(0, rows0, sem0).start()
                for j in range(CHUNK):
                    if j + 1 < CHUNK:
                        if j >= 1:
                            scatter(j - 1).wait()
                        else:
                            @pl.when(m > 0)
                            def _():
                                scatter(CHUNK - 1).wait()
                        nrows, nsem, _ = bufs[(j + 1) % 2]
                        gather(j + 1, nrows, nsem).start()
                    rows, sem, _ = bufs[j % 2]
                    gather(j, rows, sem).wait()
                    scatter(j).start()

            scatter(CHUNK - 2).wait()
            scatter(CHUNK - 1).wait()

        @pl.when(c == 0)
        def _():
            main_loop(x0_hbm)

        @pl.when(c == 1)
        def _():
            main_loop(x1_hbm)

        plsc.subcore_barrier()

        @pl.when(c == 0)
        def _():
            on_stripe(lambda r0, nr: pltpu.sync_copy(
                acc.at[pl.ds(r0, nr)], agg0_hbm.at[pl.ds(r0, nr)]))

        @pl.when(c == 1)
        def _():
            on_stripe(lambda r0, nr: pltpu.sync_copy(
                acc.at[pl.ds(r0, nr)], agg1_hbm.at[pl.ds(r0, nr)]))

    return k(x0, x1, esrc2d, edst2d, zf)


def _sc_degree(edst2d, n_dst):
    n_blocks = edst2d.shape[0]
    blocks_per_w = n_blocks // (NUM_CORES * NUM_SUBCORES)
    n_chunks = blocks_per_w // CHUNK
    n_acc = n_dst + PAD_ROWS
    stripe, last_stripe = _stripes(n_dst)

    zd = jnp.zeros((n_dst, HALF), jnp.float32)
    ones = jnp.ones((BLK, HALF), jnp.float32)

    @functools.partial(
        pl.kernel,
        out_type=(
            jax.ShapeDtypeStruct((n_dst, HALF), jnp.float32),
            jax.ShapeDtypeStruct((n_dst, HALF), jnp.float32),
        ),
        mesh=_MESH,
        scratch_types=[
            pltpu.VMEM((CHUNK, BLK), jnp.int32),
            pltpu.VMEM((BLK, HALF), jnp.float32),
            pltpu.VMEM_SHARED((n_acc, HALF), jnp.float32),
        ],
    )
    def k(ed_hbm, zd_hbm, ones_hbm, dega_hbm, degb_hbm,
          idx_d, ones_v, dacc):
        c = lax.axis_index("c")
        s = lax.axis_index("s")

        def on_stripe(fn):
            @pl.when(s < NUM_SUBCORES - 1)
            def _():
                fn(s * stripe, stripe)

            @pl.when(s == NUM_SUBCORES - 1)
            def _():
                fn((NUM_SUBCORES - 1) * stripe, last_stripe)

        blk0 = (c * NUM_SUBCORES + s) * blocks_per_w
        on_stripe(lambda r0, nr: pltpu.sync_copy(
            zd_hbm.at[pl.ds(r0, nr)], dacc.at[pl.ds(r0, nr)]))
        pltpu.sync_copy(ones_hbm, ones_v)
        plsc.subcore_barrier()

        @pl.loop(0, n_chunks)
        def _(m):
            pltpu.sync_copy(ed_hbm.at[pl.ds(blk0 + m * CHUNK, CHUNK)], idx_d)
            for j in range(CHUNK):
                pltpu.sync_copy(ones_v, dacc.at[idx_d.at[j]], add=True)

        plsc.subcore_barrier()

        @pl.when(c == 0)
        def _():
            on_stripe(lambda r0, nr: pltpu.sync_copy(
                dacc.at[pl.ds(r0, nr)], dega_hbm.at[pl.ds(r0, nr)]))

        @pl.when(c == 1)
        def _():
            on_stripe(lambda r0, nr: pltpu.sync_copy(
                dacc.at[pl.ds(r0, nr)], degb_hbm.at[pl.ds(r0, nr)]))

    return k(edst2d, zd, ones)


def _tc_mean_matmul(agg0, agg1, dega, degb, x_dst, w_nbr, w_self):
    n, d = x_dst.shape
    blk = 1000
    wn0 = w_nbr[:HALF]
    wn1 = w_nbr[HALF:]

    def body(a0_ref, a1_ref, da_ref, db_ref, x_ref,
             wn0_ref, wn1_ref, ws_ref, o_ref):
        deg = da_ref[:, 0:1] + db_ref[:, 0:1]
        r = 1.0 / jnp.maximum(deg, 1.0)
        o_ref[...] = (
            jnp.dot(a0_ref[...] * r, wn0_ref[...],
                    preferred_element_type=jnp.float32)
            + jnp.dot(a1_ref[...] * r, wn1_ref[...],
                      preferred_element_type=jnp.float32)
            + jnp.dot(x_ref[...], ws_ref[...],
                      preferred_element_type=jnp.float32)
        )

    return pl.pallas_call(
        body,
        grid=(n // blk,),
        in_specs=[
            pl.BlockSpec((blk, HALF), lambda i: (i, 0)),
            pl.BlockSpec((blk, HALF), lambda i: (i, 0)),
            pl.BlockSpec((blk, HALF), lambda i: (i, 0)),
            pl.BlockSpec((blk, HALF), lambda i: (i, 0)),
            pl.BlockSpec((blk, d), lambda i: (i, 0)),
            pl.BlockSpec((HALF, d), lambda i: (0, 0)),
            pl.BlockSpec((HALF, d), lambda i: (0, 0)),
            pl.BlockSpec((d, d), lambda i: (0, 0)),
        ],
        out_specs=pl.BlockSpec((blk, d), lambda i: (i, 0)),
        out_shape=jax.ShapeDtypeStruct((n, d), jnp.float32),
    )(agg0, agg1, dega, degb, x_dst, wn0, wn1, w_self)


def _pad_edges(e_src, e_dst, n_dst):
    e = e_src.shape[0]
    chunk = NUM_SUBCORES * BLK * CHUNK
    e_pad = ((e + chunk - 1) // chunk) * chunk
    npad = e_pad - e
    e_src = jnp.concatenate(
        [e_src.astype(jnp.int32), jnp.zeros((npad,), jnp.int32)])
    e_dst = jnp.concatenate(
        [e_dst.astype(jnp.int32), jnp.full((npad,), n_dst, jnp.int32)])
    return e_src.reshape(-1, BLK), e_dst.reshape(-1, BLK)


def _conv(x_src, x_dst, e_src, e_dst, w_nbr, w_self):
    n_dst = x_dst.shape[0]
    es2d, ed2d = _pad_edges(e_src, e_dst, n_dst)
    agg0, agg1 = _sc_gather_scatter(
        x_src[:, :HALF], x_src[:, HALF:], es2d, ed2d, n_dst)
    dega, degb = _sc_degree(ed2d, n_dst)
    return _tc_mean_matmul(agg0, agg1, dega, degb, x_dst, w_nbr, w_self)


def kernel(x_user, x_item, u2i_src, u2i_dst, i2u_src, i2u_dst,
           n_user, n_item, W_nbr_u2i, W_self_u2i, W_nbr_i2u, W_self_i2u):
    out_item = _conv(x_user, x_item, u2i_src, u2i_dst, W_nbr_u2i, W_self_u2i)
    out_user = _conv(x_item, x_user, i2u_src, i2u_dst, W_nbr_i2u, W_self_i2u)
    return (out_user, out_item)


# speedup vs baseline: 1.0707x; 1.0095x over previous
import functools

import jax
import jax.numpy as jnp
from jax import lax
from jax.experimental import pallas as pl
from jax.experimental.pallas import tpu as pltpu
from jax.experimental.pallas import tpu_sc as plsc

NUM_CORES = 2
NUM_SUBCORES = 16
BLK = 128
CHUNK = 8
HALF = 128
PAD_ROWS = 8
LANES = 16

_MESH = plsc.VectorSubcoreMesh(
    core_axis_name="c", subcore_axis_name="s",
    num_cores=NUM_CORES, num_subcores=NUM_SUBCORES)


def _stripes(n_dst):
    stripe = ((n_dst + NUM_SUBCORES - 1) // NUM_SUBCORES + 7) // 8 * 8
    return stripe, n_dst - (NUM_SUBCORES - 1) * stripe


def _sc_gather_scatter(x0, x1, esrc2d, edst2d, n_dst):
    n_blocks = esrc2d.shape[0]
    blocks_per_sub = n_blocks // NUM_SUBCORES
    n_chunks = blocks_per_sub // CHUNK
    n_acc = n_dst + PAD_ROWS
    stripe, last_stripe = _stripes(n_dst)

    zf = jnp.zeros((n_dst, HALF), jnp.float32)

    @functools.partial(
        pl.kernel,
        out_type=(
            jax.ShapeDtypeStruct((n_dst, HALF), jnp.float32),
            jax.ShapeDtypeStruct((n_dst, HALF), jnp.float32),
        ),
        mesh=_MESH,
        scratch_types=[
            pltpu.VMEM((2, CHUNK, BLK), jnp.int32),
            pltpu.VMEM((2, CHUNK, BLK), jnp.int32),
            pltpu.VMEM((BLK, HALF), jnp.float32),
            pltpu.VMEM((BLK, HALF), jnp.float32),
            pltpu.VMEM_SHARED((n_acc, HALF), jnp.float32),
            pltpu.SemaphoreType.DMA,
            pltpu.SemaphoreType.DMA,
            pltpu.SemaphoreType.DMA,
            pltpu.SemaphoreType.DMA,
            pltpu.SemaphoreType.DMA,
        ],
    )
    def k(x0_hbm, x1_hbm, es_hbm, ed_hbm, zf_hbm,
          agg0_hbm, agg1_hbm,
          idx_s, idx_d, rows0, rows1, acc,
          sem0, sem1, ssem0, ssem1, sem_i):
        c = lax.axis_index("c")
        s = lax.axis_index("s")

        def on_stripe(fn):
            @pl.when(s < NUM_SUBCORES - 1)
            def _():
                fn(s * stripe, stripe)

            @pl.when(s == NUM_SUBCORES - 1)
            def _():
                fn((NUM_SUBCORES - 1) * stripe, last_stripe)

        blk0 = s * blocks_per_sub
        on_stripe(lambda r0, nr: pltpu.sync_copy(
            zf_hbm.at[pl.ds(r0, nr)], acc.at[pl.ds(r0, nr)]))
        plsc.subcore_barrier()

        bufs = ((rows0, sem0, ssem0), (rows1, sem1, ssem1))

        def idx_load(m, sl):
            return (
                pltpu.make_async_copy(
                    es_hbm.at[pl.ds(blk0 + m * CHUNK, CHUNK)],
                    idx_s.at[sl], sem_i),
                pltpu.make_async_copy(
                    ed_hbm.at[pl.ds(blk0 + m * CHUNK, CHUNK)],
                    idx_d.at[sl], sem_i),
            )

        def main_loop(x_hbm):
            def gather(sl, j, rows, sem):
                return pltpu.make_async_copy(x_hbm.at[idx_s.at[sl].at[j]],
                                             rows, sem)

            class scatter:
                def __init__(self, sl, j):
                    self.rows, _, self.ssem = bufs[j % 2]
                    self.sl, self.j = sl, j

                def start(self):
                    pltpu.async_copy(
                        self.rows, acc.at[idx_d.at[self.sl].at[self.j]],
                        self.ssem, add=True)

                def wait(self):
                    pltpu.make_async_copy(
                        self.rows, acc.at[idx_d.at[self.sl].at[self.j]],
                        self.ssem).wait()

            for d in idx_load(0, 0):
                d.start()

            @pl.loop(0, n_chunks)
            def _(m):
                sl = lax.rem(m, 2)
                nsl = 1 - sl
                for d in idx_load(m, sl):
                    d.wait()

                @pl.when(m > 0)
                def _():
                    scatter(nsl, CHUNK - 2).wait()
                    scatter(nsl, CHUNK - 1).wait()

                @pl.when(m + 1 < n_chunks)
                def _():
                    for d in idx_load(m + 1, nsl):
                        d.start()

                gather(sl, 0, rows0, sem0).start()
                for j in range(CHUNK):
                    if j + 1 < CHUNK:
                        if j >= 1:
                            scatter(sl, j - 1).wait()
                        nrows, nsem, _ = bufs[(j + 1) % 2]
                        gather(sl, j + 1, nrows, nsem).start()
                    rows, sem, _ = bufs[j % 2]
                    gather(sl, j, rows, sem).wait()
                    scatter(sl, j).start()

            last = (n_chunks - 1) % 2
            scatter(last, CHUNK - 2).wait()
            scatter(last, CHUNK - 1).wait()

        @pl.when(c == 0)
        def _():
            main_loop(x0_hbm)

        @pl.when(c == 1)
        def _():
            main_loop(x1_hbm)

        plsc.subcore_barrier()

        @pl.when(c == 0)
        def _():
            on_stripe(lambda r0, nr: pltpu.sync_copy(
                acc.at[pl.ds(r0, nr)], agg0_hbm.at[pl.ds(r0, nr)]))

        @pl.when(c == 1)
        def _():
            on_stripe(lambda r0, nr: pltpu.sync_copy(
                acc.at[pl.ds(r0, nr)], agg1_hbm.at[pl.ds(r0, nr)]))

    return k(x0, x1, esrc2d, edst2d, zf)


def _sc_degree(edst2d, n_dst):
    n_blocks = edst2d.shape[0]
    blocks_per_w = n_blocks // (NUM_CORES * NUM_SUBCORES)
    n_chunks = blocks_per_w // CHUNK
    n_acc = n_dst + PAD_ROWS
    stripe, last_stripe = _stripes(n_dst)

    zd = jnp.zeros((n_dst, HALF), jnp.float32)
    ones = jnp.ones((BLK, HALF), jnp.float32)

    @functools.partial(
        pl.kernel,
        out_type=(
            jax.ShapeDtypeStruct((n_dst, HALF), jnp.float32),
            jax.ShapeDtypeStruct((n_dst, HALF), jnp.float32),
        ),
        mesh=_MESH,
        scratch_types=[
            pltpu.VMEM((CHUNK, BLK), jnp.int32),
            pltpu.VMEM((BLK, HALF), jnp.float32),
            pltpu.VMEM_SHARED((n_acc, HALF), jnp.float32),
        ],
    )
    def k(ed_hbm, zd_hbm, ones_hbm, dega_hbm, degb_hbm,
          idx_d, ones_v, dacc):
        c = lax.axis_index("c")
        s = lax.axis_index("s")

        def on_stripe(fn):
            @pl.when(s < NUM_SUBCORES - 1)
            def _():
                fn(s * stripe, stripe)

            @pl.when(s == NUM_SUBCORES - 1)
            def _():
                fn((NUM_SUBCORES - 1) * stripe, last_stripe)

        blk0 = (c * NUM_SUBCORES + s) * blocks_per_w
        on_stripe(lambda r0, nr: pltpu.sync_copy(
            zd_hbm.at[pl.ds(r0, nr)], dacc.at[pl.ds(r0, nr)]))
        pltpu.sync_copy(ones_hbm, ones_v)
        plsc.subcore_barrier()

        @pl.loop(0, n_chunks)
        def _(m):
            pltpu.sync_copy(ed_hbm.at[pl.ds(blk0 + m * CHUNK, CHUNK)], idx_d)
            for j in range(CHUNK):
                pltpu.sync_copy(ones_v, dacc.at[idx_d.at[j]], add=True)

        plsc.subcore_barrier()

        @pl.when(c == 0)
        def _():
            on_stripe(lambda r0, nr: pltpu.sync_copy(
                dacc.at[pl.ds(r0, nr)], dega_hbm.at[pl.ds(r0, nr)]))

        @pl.when(c == 1)
        def _():
            on_stripe(lambda r0, nr: pltpu.sync_copy(
                dacc.at[pl.ds(r0, nr)], degb_hbm.at[pl.ds(r0, nr)]))

    return k(edst2d, zd, ones)


def _tc_mean_matmul(agg0, agg1, dega, degb, x_dst, w_nbr, w_self):
    n, d = x_dst.shape
    blk = 1000
    wn0 = w_nbr[:HALF]
    wn1 = w_nbr[HALF:]

    def body(a0_ref, a1_ref, da_ref, db_ref, x_ref,
             wn0_ref, wn1_ref, ws_ref, o_ref):
        deg = da_ref[:, 0:1] + db_ref[:, 0:1]
        r = 1.0 / jnp.maximum(deg, 1.0)
        o_ref[...] = (
            jnp.dot(a0_ref[...] * r, wn0_ref[...],
                    preferred_element_type=jnp.float32)
            + jnp.dot(a1_ref[...] * r, wn1_ref[...],
                      preferred_element_type=jnp.float32)
            + jnp.dot(x_ref[...], ws_ref[...],
                      preferred_element_type=jnp.float32)
        )

    return pl.pallas_call(
        body,
        grid=(n // blk,),
        in_specs=[
            pl.BlockSpec((blk, HALF), lambda i: (i, 0)),
            pl.BlockSpec((blk, HALF), lambda i: (i, 0)),
            pl.BlockSpec((blk, HALF), lambda i: (i, 0)),
            pl.BlockSpec((blk, HALF), lambda i: (i, 0)),
            pl.BlockSpec((blk, d), lambda i: (i, 0)),
            pl.BlockSpec((HALF, d), lambda i: (0, 0)),
            pl.BlockSpec((HALF, d), lambda i: (0, 0)),
            pl.BlockSpec((d, d), lambda i: (0, 0)),
        ],
        out_specs=pl.BlockSpec((blk, d), lambda i: (i, 0)),
        out_shape=jax.ShapeDtypeStruct((n, d), jnp.float32),
    )(agg0, agg1, dega, degb, x_dst, wn0, wn1, w_self)


def _pad_edges(e_src, e_dst, n_dst):
    e = e_src.shape[0]
    chunk = NUM_SUBCORES * BLK * CHUNK
    e_pad = ((e + chunk - 1) // chunk) * chunk
    npad = e_pad - e
    e_src = jnp.concatenate(
        [e_src.astype(jnp.int32), jnp.zeros((npad,), jnp.int32)])
    e_dst = jnp.concatenate(
        [e_dst.astype(jnp.int32), jnp.full((npad,), n_dst, jnp.int32)])
    return e_src.reshape(-1, BLK), e_dst.reshape(-1, BLK)


def _conv(x_src, x_dst, e_src, e_dst, w_nbr, w_self):
    n_dst = x_dst.shape[0]
    es2d, ed2d = _pad_edges(e_src, e_dst, n_dst)
    agg0, agg1 = _sc_gather_scatter(
        x_src[:, :HALF], x_src[:, HALF:], es2d, ed2d, n_dst)
    dega, degb = _sc_degree(ed2d, n_dst)
    return _tc_mean_matmul(agg0, agg1, dega, degb, x_dst, w_nbr, w_self)


def kernel(x_user, x_item, u2i_src, u2i_dst, i2u_src, i2u_dst,
           n_user, n_item, W_nbr_u2i, W_self_u2i, W_nbr_i2u, W_self_i2u):
    out_item = _conv(x_user, x_item, u2i_src, u2i_dst, W_nbr_u2i, W_self_u2i)
    out_user = _conv(x_item, x_user, i2u_src, i2u_dst, W_nbr_i2u, W_self_i2u)
    return (out_user, out_item)


# speedup vs baseline: 1.1099x; 1.0366x over previous
import functools

import jax
import jax.numpy as jnp
from jax import lax
from jax.experimental import pallas as pl
from jax.experimental.pallas import tpu as pltpu
from jax.experimental.pallas import tpu_sc as plsc

NUM_CORES = 2
NUM_SUBCORES = 16
BLK = 128
CHUNK = 8
HALF = 128
PAD_ROWS = 8
LANES = 16

_MESH = plsc.VectorSubcoreMesh(
    core_axis_name="c", subcore_axis_name="s",
    num_cores=NUM_CORES, num_subcores=NUM_SUBCORES)


def _stripes(n_dst):
    stripe = ((n_dst + NUM_SUBCORES - 1) // NUM_SUBCORES + 7) // 8 * 8
    return stripe, n_dst - (NUM_SUBCORES - 1) * stripe


def _sc_gather_scatter(x0a, x1a, esa, eda, x0b, x1b, esb, edb, n_dst):
    n_blocks = esa.shape[0]
    blocks_per_sub = n_blocks // NUM_SUBCORES
    n_chunks = blocks_per_sub // CHUNK
    n_acc = n_dst + PAD_ROWS
    stripe, last_stripe = _stripes(n_dst)

    zf = jnp.zeros((n_dst, HALF), jnp.float32)
    out = jax.ShapeDtypeStruct((n_dst, HALF), jnp.float32)

    @functools.partial(
        pl.kernel,
        out_type=(out, out, out, out),
        mesh=_MESH,
        scratch_types=[
            pltpu.VMEM((2, CHUNK, BLK), jnp.int32),
            pltpu.VMEM((2, CHUNK, BLK), jnp.int32),
            pltpu.VMEM((BLK, HALF), jnp.float32),
            pltpu.VMEM((BLK, HALF), jnp.float32),
            pltpu.VMEM_SHARED((n_acc, HALF), jnp.float32),
            pltpu.SemaphoreType.DMA,
            pltpu.SemaphoreType.DMA,
            pltpu.SemaphoreType.DMA,
            pltpu.SemaphoreType.DMA,
            pltpu.SemaphoreType.DMA,
        ],
    )
    def k(x0a_hbm, x1a_hbm, esa_hbm, eda_hbm,
          x0b_hbm, x1b_hbm, esb_hbm, edb_hbm, zf_hbm,
          agg0a_hbm, agg1a_hbm, agg0b_hbm, agg1b_hbm,
          idx_s, idx_d, rows0, rows1, acc,
          sem0, sem1, ssem0, ssem1, sem_i):
        c = lax.axis_index("c")
        s = lax.axis_index("s")

        def on_stripe(fn):
            @pl.when(s < NUM_SUBCORES - 1)
            def _():
                fn(s * stripe, stripe)

            @pl.when(s == NUM_SUBCORES - 1)
            def _():
                fn((NUM_SUBCORES - 1) * stripe, last_stripe)

        blk0 = s * blocks_per_sub
        bufs = ((rows0, sem0, ssem0), (rows1, sem1, ssem1))

        def idx_load(es_hbm, ed_hbm, m, sl):
            return (
                pltpu.make_async_copy(
                    es_hbm.at[pl.ds(blk0 + m * CHUNK, CHUNK)],
                    idx_s.at[sl], sem_i),
                pltpu.make_async_copy(
                    ed_hbm.at[pl.ds(blk0 + m * CHUNK, CHUNK)],
                    idx_d.at[sl], sem_i),
            )

        def main_loop(x_hbm, es_hbm, ed_hbm):
            def gather(sl, j, rows, sem):
                return pltpu.make_async_copy(x_hbm.at[idx_s.at[sl].at[j]],
                                             rows, sem)

            class scatter:
                def __init__(self, sl, j):
                    self.rows, _, self.ssem = bufs[j % 2]
                    self.sl, self.j = sl, j

                def start(self):
                    pltpu.async_copy(
                        self.rows, acc.at[idx_d.at[self.sl].at[self.j]],
                        self.ssem, add=True)

                def wait(self):
                    pltpu.make_async_copy(
                        self.rows, acc.at[idx_d.at[self.sl].at[self.j]],
                        self.ssem).wait()

            for d in idx_load(es_hbm, ed_hbm, 0, 0):
                d.start()

            @pl.loop(0, n_chunks)
            def _(m):
                sl = lax.rem(m, 2)
                nsl = 1 - sl
                for d in idx_load(es_hbm, ed_hbm, m, sl):
                    d.wait()

                @pl.when(m > 0)
                def _():
                    scatter(nsl, CHUNK - 2).wait()
                    scatter(nsl, CHUNK - 1).wait()

                @pl.when(m + 1 < n_chunks)
                def _():
                    for d in idx_load(es_hbm, ed_hbm, m + 1, nsl):
                        d.start()

                gather(sl, 0, rows0, sem0).start()
                for j in range(CHUNK):
                    if j + 1 < CHUNK:
                        if j >= 1:
                            scatter(sl, j - 1).wait()
                        nrows, nsem, _ = bufs[(j + 1) % 2]
                        gather(sl, j + 1, nrows, nsem).start()
                    rows, sem, _ = bufs[j % 2]
                    gather(sl, j, rows, sem).wait()
                    scatter(sl, j).start()

            last = (n_chunks - 1) % 2
            scatter(last, CHUNK - 2).wait()
            scatter(last, CHUNK - 1).wait()

        def phase(x0_hbm, x1_hbm, es_hbm, ed_hbm, out0_hbm, out1_hbm):
            on_stripe(lambda r0, nr: pltpu.sync_copy(
                zf_hbm.at[pl.ds(r0, nr)], acc.at[pl.ds(r0, nr)]))
            plsc.subcore_barrier()

            @pl.when(c == 0)
            def _():
                main_loop(x0_hbm, es_hbm, ed_hbm)

            @pl.when(c == 1)
            def _():
                main_loop(x1_hbm, es_hbm, ed_hbm)

            plsc.subcore_barrier()

            @pl.when(c == 0)
            def _():
                on_stripe(lambda r0, nr: pltpu.sync_copy(
                    acc.at[pl.ds(r0, nr)], out0_hbm.at[pl.ds(r0, nr)]))

            @pl.when(c == 1)
            def _():
                on_stripe(lambda r0, nr: pltpu.sync_copy(
                    acc.at[pl.ds(r0, nr)], out1_hbm.at[pl.ds(r0, nr)]))

        phase(x0a_hbm, x1a_hbm, esa_hbm, eda_hbm, agg0a_hbm, agg1a_hbm)
        phase(x0b_hbm, x1b_hbm, esb_hbm, edb_hbm, agg0b_hbm, agg1b_hbm)

    return k(x0a, x1a, esa, eda, x0b, x1b, esb, edb, zf)


def _sc_degree(eda, edb, n_dst):
    n_blocks = eda.shape[0]
    blocks_per_w = n_blocks // (NUM_CORES * NUM_SUBCORES)
    n_chunks = blocks_per_w // CHUNK
    n_acc = n_dst + PAD_ROWS
    stripe, last_stripe = _stripes(n_dst)

    zd = jnp.zeros((n_dst, HALF), jnp.float32)
    ones = jnp.ones((BLK, HALF), jnp.float32)
    out = jax.ShapeDtypeStruct((n_dst, HALF), jnp.float32)

    @functools.partial(
        pl.kernel,
        out_type=(out, out, out, out),
        mesh=_MESH,
        scratch_types=[
            pltpu.VMEM((CHUNK, BLK), jnp.int32),
            pltpu.VMEM((BLK, HALF), jnp.float32),
            pltpu.VMEM_SHARED((n_acc, HALF), jnp.float32),
        ],
    )
    def k(eda_hbm, edb_hbm, zd_hbm, ones_hbm,
          degaa_hbm, degba_hbm, degab_hbm, degbb_hbm,
          idx_d, ones_v, dacc):
        c = lax.axis_index("c")
        s = lax.axis_index("s")

        def on_stripe(fn):
            @pl.when(s < NUM_SUBCORES - 1)
            def _():
                fn(s * stripe, stripe)

            @pl.when(s == NUM_SUBCORES - 1)
            def _():
                fn((NUM_SUBCORES - 1) * stripe, last_stripe)

        blk0 = (c * NUM_SUBCORES + s) * blocks_per_w
        pltpu.sync_copy(ones_hbm, ones_v)

        def phase(ed_hbm, dega_hbm, degb_hbm):
            on_stripe(lambda r0, nr: pltpu.sync_copy(
                zd_hbm.at[pl.ds(r0, nr)], dacc.at[pl.ds(r0, nr)]))
            plsc.subcore_barrier()

            @pl.loop(0, n_chunks)
            def _(m):
                pltpu.sync_copy(ed_hbm.at[pl.ds(blk0 + m * CHUNK, CHUNK)],
                                idx_d)
                for j in range(CHUNK):
                    pltpu.sync_copy(ones_v, dacc.at[idx_d.at[j]], add=True)

            plsc.subcore_barrier()

            @pl.when(c == 0)
            def _():
                on_stripe(lambda r0, nr: pltpu.sync_copy(
                    dacc.at[pl.ds(r0, nr)], dega_hbm.at[pl.ds(r0, nr)]))

            @pl.when(c == 1)
            def _():
                on_stripe(lambda r0, nr: pltpu.sync_copy(
                    dacc.at[pl.ds(r0, nr)], degb_hbm.at[pl.ds(r0, nr)]))

        phase(eda_hbm, degaa_hbm, degba_hbm)
        phase(edb_hbm, degab_hbm, degbb_hbm)

    return k(eda, edb, zd, ones)


def _tc_mean_matmul(agg0, agg1, dega, degb, x_dst, w_nbr, w_self):
    n, d = x_dst.shape
    blk = 1000
    wn0 = w_nbr[:HALF]
    wn1 = w_nbr[HALF:]

    def body(a0_ref, a1_ref, da_ref, db_ref, x_ref,
             wn0_ref, wn1_ref, ws_ref, o_ref):
        deg = da_ref[:, 0:1] + db_ref[:, 0:1]
        r = 1.0 / jnp.maximum(deg, 1.0)
        o_ref[...] = (
            jnp.dot(a0_ref[...] * r, wn0_ref[...],
                    preferred_element_type=jnp.float32)
            + jnp.dot(a1_ref[...] * r, wn1_ref[...],
                      preferred_element_type=jnp.float32)
            + jnp.dot(x_ref[...], ws_ref[...],
                      preferred_element_type=jnp.float32)
        )

    return pl.pallas_call(
        body,
        grid=(n // blk,),
        in_specs=[
            pl.BlockSpec((blk, HALF), lambda i: (i, 0)),
            pl.BlockSpec((blk, HALF), lambda i: (i, 0)),
            pl.BlockSpec((blk, HALF), lambda i: (i, 0)),
            pl.BlockSpec((blk, HALF), lambda i: (i, 0)),
            pl.BlockSpec((blk, d), lambda i: (i, 0)),
            pl.BlockSpec((HALF, d), lambda i: (0, 0)),
            pl.BlockSpec((HALF, d), lambda i: (0, 0)),
            pl.BlockSpec((d, d), lambda i: (0, 0)),
        ],
        out_specs=pl.BlockSpec((blk, d), lambda i: (i, 0)),
        out_shape=jax.ShapeDtypeStruct((n, d), jnp.float32),
    )(agg0, agg1, dega, degb, x_dst, wn0, wn1, w_self)


def _pad_edges(e_src, e_dst, n_dst):
    e = e_src.shape[0]
    chunk = NUM_SUBCORES * BLK * CHUNK
    e_pad = ((e + chunk - 1) // chunk) * chunk
    npad = e_pad - e
    e_src = jnp.concatenate(
        [e_src.astype(jnp.int32), jnp.zeros((npad,), jnp.int32)])
    e_dst = jnp.concatenate(
        [e_dst.astype(jnp.int32), jnp.full((npad,), n_dst, jnp.int32)])
    return e_src.reshape(-1, BLK), e_dst.reshape(-1, BLK)


def kernel(x_user, x_item, u2i_src, u2i_dst, i2u_src, i2u_dst,
           n_user, n_item, W_nbr_u2i, W_self_u2i, W_nbr_i2u, W_self_i2u):
    n_item_s = x_item.shape[0]
    n_user_s = x_user.shape[0]
    esa, eda = _pad_edges(u2i_src, u2i_dst, n_item_s)
    esb, edb = _pad_edges(i2u_src, i2u_dst, n_user_s)
    agg0a, agg1a, agg0b, agg1b = _sc_gather_scatter(
        x_user[:, :HALF], x_user[:, HALF:], esa, eda,
        x_item[:, :HALF], x_item[:, HALF:], esb, edb, n_item_s)
    degaa, degba, degab, degbb = _sc_degree(eda, edb, n_item_s)
    out_item = _tc_mean_matmul(agg0a, agg1a, degaa, degba,
                               x_item, W_nbr_u2i, W_self_u2i)
    out_user = _tc_mean_matmul(agg0b, agg1b, degab, degbb,
                               x_user, W_nbr_i2u, W_self_i2u)
    return (out_user, out_item)
